# Initial kernel scaffold; baseline (speedup 1.0000x reference)
import jax
import jax.numpy as jnp
from jax.experimental import pallas as pl


def kernel(x, edge_index, batch, W1, b1, W2, b2, W3, b3, fW1, fb1, fW2, fb2):
    raise NotImplementedError("write your pallas kernel here")


# speedup vs baseline: 4.6219x; 4.6219x over previous
import functools

import jax
import jax.numpy as jnp
from jax import lax
from jax.experimental import pallas as pl
from jax.experimental.pallas import tpu as pltpu
from jax.experimental.pallas import tpu_sc as plsc

N = 10000
E = 320000
D = 128
NCLS = 16
NG = 64

NPAD = 10240
DUMMY = N
C = 128
ECHUNKS = 2560
EPAD = ECHUNKS * C
CPT = ECHUNKS // 16
PH = 80
HALF = NPAD // 2
ACC_R = HALF + 256
ATRASH = HALF
OPT = HALF // 16
PCHUNKS = NPAD // C
POOL_ROWS = 256
TRASH = 128


def _fill_rows(ref, nrows, value):
  vec = jnp.full((16,), value, jnp.float32)

  def body(i, carry):
    for cc in range(8):
      ref[i, pl.ds(cc * 16, 16)] = vec
    return carry

  lax.fori_loop(0, nrows, body, 0)


def _local_idx(dst_v, k, base, lidx):
  for cc in range(8):
    d = dst_v[k, pl.ds(cc * 16, 16)]
    ld = d - base
    ok = (ld >= 0) & (ld < HALF)
    lidx[0, pl.ds(cc * 16, 16)] = jnp.where(ok, ld, ATRASH)


def _zero_acc_slice(zeros_v, acc, sid):
  rows = ACC_R // 16
  base = sid * rows
  pltpu.sync_copy(zeros_v, acc.at[pl.ds(base, C)])
  pltpu.sync_copy(zeros_v, acc.at[pl.ds(base + C, C)])
  pltpu.sync_copy(zeros_v.at[pl.ds(0, rows - 2 * C)],
                  acc.at[pl.ds(base + 2 * C, rows - 2 * C)])


def _acc_to_out(acc, out_hbm, cid, sid):
  for o, sz in ((0, C), (C, C), (2 * C, OPT - 2 * C)):
    pltpu.sync_copy(acc.at[pl.ds(sid * OPT + o, sz)],
                    out_hbm.at[pl.ds(cid * HALF + sid * OPT + o, sz)])


def _make_deg_kernel(mesh):
  @functools.partial(
      pl.kernel,
      out_type=jax.ShapeDtypeStruct((NPAD, D), jnp.float32),
      mesh=mesh,
      scratch_types=[
          pltpu.VMEM((CPT, C), jnp.int32),
          pltpu.VMEM((1, C), jnp.int32),
          pltpu.VMEM((C, D), jnp.float32),
          pltpu.VMEM((C, D), jnp.float32),
          pltpu.VMEM_SHARED((ACC_R, D), jnp.float32),
      ],
  )
  def deg_kernel(dst_hbm, out_hbm, dst_v, lidx, ones_v, zeros_v, acc):
    cid = lax.axis_index("c")
    sid = lax.axis_index("s")
    base = cid * HALF
    pltpu.sync_copy(dst_hbm.at[pl.ds(sid * CPT, CPT)], dst_v)
    _fill_rows(ones_v, C, 1.0)
    _fill_rows(zeros_v, C, 0.0)
    _zero_acc_slice(zeros_v, acc, sid)
    plsc.subcore_barrier()

    def body(k, carry):
      _local_idx(dst_v, k, base, lidx)
      pltpu.sync_copy(ones_v, acc.at[lidx.at[0]], add=True)
      return carry

    lax.fori_loop(0, CPT, body, 0)
    plsc.subcore_barrier()
    _acc_to_out(acc, out_hbm, cid, sid)

  return deg_kernel


def _make_agg_kernel(mesh):
  @functools.partial(
      pl.kernel,
      out_type=jax.ShapeDtypeStruct((NPAD, D), jnp.float32),
      mesh=mesh,
      scratch_types=[
          pltpu.VMEM((PH, C), jnp.int32),
          pltpu.VMEM((PH, C), jnp.int32),
          pltpu.VMEM((1, C), jnp.int32),
          pltpu.VMEM((C, D), jnp.float32),
          pltpu.VMEM((C, D), jnp.float32),
          pltpu.VMEM((C, D), jnp.float32),
          pltpu.SemaphoreType.DMA,
          pltpu.SemaphoreType.DMA,
          pltpu.VMEM_SHARED((ACC_R, D), jnp.float32),
      ],
  )
  def agg_kernel(hp_hbm, src_hbm, dst_hbm, out_hbm,
                 src_v, dst_v, lidx, rows0, rows1, zeros_v, g0, g1, acc):
    cid = lax.axis_index("c")
    sid = lax.axis_index("s")
    base = cid * HALF
    _fill_rows(zeros_v, C, 0.0)
    _zero_acc_slice(zeros_v, acc, sid)
    plsc.subcore_barrier()

    for p in range(CPT // PH):
      pltpu.sync_copy(src_hbm.at[pl.ds(sid * CPT + p * PH, PH)], src_v)
      pltpu.sync_copy(dst_hbm.at[pl.ds(sid * CPT + p * PH, PH)], dst_v)
      pltpu.make_async_copy(hp_hbm.at[src_v.at[0]], rows0, g0).start()

      def body(t, carry):
        k0 = 2 * t
        k1 = k0 + 1
        pltpu.make_async_copy(hp_hbm.at[src_v.at[k0]], rows0, g0).wait()
        pltpu.make_async_copy(hp_hbm.at[src_v.at[k1]], rows1, g1).start()
        _local_idx(dst_v, k0, base, lidx)
        pltpu.sync_copy(rows0, acc.at[lidx.at[0]], add=True)
        pltpu.make_async_copy(hp_hbm.at[src_v.at[k1]], rows1, g1).wait()

        @pl.when(t < PH // 2 - 1)
        def _():
          pltpu.make_async_copy(hp_hbm.at[src_v.at[k0 + 2]], rows0, g0).start()

        _local_idx(dst_v, k1, base, lidx)
        pltpu.sync_copy(rows1, acc.at[lidx.at[0]], add=True)
        return carry

      lax.fori_loop(0, PH // 2, body, 0)
    plsc.subcore_barrier()
    _acc_to_out(acc, out_hbm, cid, sid)

  return agg_kernel


def _make_pool_kernel(mesh):
  @functools.partial(
      pl.kernel,
      out_type=jax.ShapeDtypeStruct((2, POOL_ROWS // 2, D), jnp.float32),
      mesh=mesh,
      scratch_types=[
          pltpu.VMEM((1, C), jnp.int32),
          pltpu.VMEM((1, C), jnp.int32),
          pltpu.VMEM((C, D), jnp.float32),
          pltpu.VMEM((C, D), jnp.float32),
          pltpu.VMEM((16, D), jnp.float32),
          pltpu.VMEM_SHARED((POOL_ROWS, D), jnp.float32),
      ],
  )
  def pool_kernel(h3_hbm, bat_hbm, bat64_hbm, out_hbm,
                  bidx, b64idx, rows_v, ones_v, zeros_v, acc):
    cid = lax.axis_index("c")
    sid = lax.axis_index("s")
    w = sid * 2 + cid
    _fill_rows(ones_v, C, 1.0)
    _fill_rows(zeros_v, 16, 0.0)
    pltpu.sync_copy(zeros_v, acc.at[pl.ds(sid * 16, 16)])
    plsc.subcore_barrier()
    for j in range(3):
      c = w + 32 * j

      @pl.when(c < PCHUNKS)
      def _():
        pltpu.sync_copy(bat_hbm.at[pl.ds(c, 1)], bidx)
        pltpu.sync_copy(bat64_hbm.at[pl.ds(c, 1)], b64idx)
        pltpu.sync_copy(h3_hbm.at[pl.ds(c * C, C)], rows_v)
        pltpu.sync_copy(rows_v, acc.at[bidx.at[0]], add=True)
        pltpu.sync_copy(ones_v, acc.at[b64idx.at[0]], add=True)

    plsc.subcore_barrier()
    pltpu.sync_copy(acc.at[pl.ds(sid * 8, 8)],
                    out_hbm.at[cid, pl.ds(sid * 8, 8)])

  return pool_kernel


def _tc_pre_body(x_ref, w_ref, degp_ref, hp_ref, dinv_ref):
  deg = degp_ref[...] + 1.0
  dinv = lax.rsqrt(deg)
  z = jnp.dot(x_ref[...], w_ref[...], preferred_element_type=jnp.float32)
  hp_ref[...] = dinv * z
  dinv_ref[...] = dinv[:, 0:1]


def _tc_mid_body(p_ref, hp_ref, dinv_ref, b_ref, w_ref, out_ref):
  agg = p_ref[...] + hp_ref[...]
  h = jnp.maximum(dinv_ref[...] * agg + b_ref[...], 0.0)
  out_ref[...] = dinv_ref[...] * jnp.dot(
      h, w_ref[...], preferred_element_type=jnp.float32)


def _tc_post_body(p_ref, hp_ref, dinv_ref, b_ref, out_ref):
  agg = p_ref[...] + hp_ref[...]
  out_ref[...] = jnp.maximum(dinv_ref[...] * agg + b_ref[...], 0.0)


def _tc_head_body(pp_ref, fw1_ref, fb1_ref, fw2_ref, fb2_ref, out_ref):
  p = pp_ref[0] + pp_ref[1]
  s = p[0:NG, :]
  cnt = p[NG:2 * NG, 0:1]
  g = s / jnp.maximum(cnt, 1.0)
  r = jnp.maximum(
      jnp.dot(g, fw1_ref[...], preferred_element_type=jnp.float32)
      + fb1_ref[...], 0.0)
  out_ref[...] = jnp.dot(
      r, fw2_ref[...], preferred_element_type=jnp.float32) + fb2_ref[...]


_R = 1024
_G = NPAD // _R


def _row_spec(bl=_R):
  return pl.BlockSpec((bl, D), lambda i: (i, 0))


def _tc_pre(xp, W1, degp):
  return pl.pallas_call(
      _tc_pre_body,
      grid=(_G,),
      in_specs=[
          _row_spec(),
          pl.BlockSpec((D, D), lambda i: (0, 0)),
          _row_spec(),
      ],
      out_specs=[
          _row_spec(),
          pl.BlockSpec((_R, 1), lambda i: (i, 0)),
      ],
      out_shape=[
          jax.ShapeDtypeStruct((NPAD, D), jnp.float32),
          jax.ShapeDtypeStruct((NPAD, 1), jnp.float32),
      ],
  )(xp, W1, degp)


def _tc_mid(parts, hp, dinv, b, W):
  return pl.pallas_call(
      _tc_mid_body,
      grid=(_G,),
      in_specs=[
          _row_spec(),
          _row_spec(),
          pl.BlockSpec((_R, 1), lambda i: (i, 0)),
          pl.BlockSpec((1, D), lambda i: (0, 0)),
          pl.BlockSpec((D, D), lambda i: (0, 0)),
      ],
      out_specs=_row_spec(),
      out_shape=jax.ShapeDtypeStruct((NPAD, D), jnp.float32),
  )(parts, hp, dinv, b, W)


def _tc_post(parts, hp, dinv, b):
  return pl.pallas_call(
      _tc_post_body,
      grid=(_G,),
      in_specs=[
          _row_spec(),
          _row_spec(),
          pl.BlockSpec((_R, 1), lambda i: (i, 0)),
          pl.BlockSpec((1, D), lambda i: (0, 0)),
      ],
      out_specs=_row_spec(),
      out_shape=jax.ShapeDtypeStruct((NPAD, D), jnp.float32),
  )(parts, hp, dinv, b)


def _tc_head(pool, fW1, fb1, fW2, fb2):
  return pl.pallas_call(
      _tc_head_body,
      out_shape=jax.ShapeDtypeStruct((NG, NCLS), jnp.float32),
  )(pool, fW1, fb1, fW2, fb2)


def kernel(x, edge_index, batch, W1, b1, W2, b2, W3, b3, fW1, fb1, fW2, fb2):
  mesh = plsc.VectorSubcoreMesh(core_axis_name="c", subcore_axis_name="s")
  deg_k = _make_deg_kernel(mesh)
  agg_k = _make_agg_kernel(mesh)
  pool_k = _make_pool_kernel(mesh)

  src = edge_index[0].astype(jnp.int32)
  dst = edge_index[1].astype(jnp.int32)
  pad = jnp.full((EPAD - E,), DUMMY, jnp.int32)
  srcp = jnp.concatenate([src, pad]).reshape(ECHUNKS, C)
  dstp = jnp.concatenate([dst, pad]).reshape(ECHUNKS, C)
  xp = jnp.zeros((NPAD, D), jnp.float32).at[:N].set(x)
  batp = jnp.concatenate(
      [batch.astype(jnp.int32), jnp.full((NPAD - N,), TRASH, jnp.int32)]
  ).reshape(PCHUNKS, C)
  batp64 = batp + NG

  degp = deg_k(dstp)
  hp1, dinv = _tc_pre(xp, W1, degp)
  parts1 = agg_k(hp1, srcp, dstp)
  hp2 = _tc_mid(parts1, hp1, dinv, b1.reshape(1, D), W2)
  parts2 = agg_k(hp2, srcp, dstp)
  hp3 = _tc_mid(parts2, hp2, dinv, b2.reshape(1, D), W3)
  parts3 = agg_k(hp3, srcp, dstp)
  h3 = _tc_post(parts3, hp3, dinv, b3.reshape(1, D))
  pool = pool_k(h3, batp, batp64)
  return _tc_head(pool, fW1, fb1.reshape(1, D), fW2, fb2.reshape(1, NCLS))


# speedup vs baseline: 4.8357x; 1.0463x over previous
import functools

import jax
import jax.numpy as jnp
from jax import lax
from jax.experimental import pallas as pl
from jax.experimental.pallas import tpu as pltpu
from jax.experimental.pallas import tpu_sc as plsc

N = 10000
E = 320000
D = 128
NCLS = 16
NG = 64

NPAD = 10240
DUMMY = N
C = 128
ECHUNKS = 2560
EPAD = ECHUNKS * C
CPT = ECHUNKS // 16
PH = 80
HALF = NPAD // 2
ACC_R = HALF + 256
ATRASH = HALF
OPT = HALF // 16
PCHUNKS = NPAD // C
POOL_ROWS = 256
TRASH = 128


def _fill_rows(ref, nrows, value):
  vec = jnp.full((16,), value, jnp.float32)

  def body(i, carry):
    for cc in range(8):
      ref[i, pl.ds(cc * 16, 16)] = vec
    return carry

  lax.fori_loop(0, nrows, body, 0)


def _local_idx(dst_v, k, base, lidx):
  lane = lax.iota(jnp.int32, 16)
  for cc in range(8):
    d = dst_v[k, pl.ds(cc * 16, 16)]
    ld = d - base
    ok = (ld >= 0) & (ld < HALF)
    lidx[0, pl.ds(cc * 16, 16)] = jnp.where(ok, ld, ATRASH + cc * 16 + lane)


def _zero_acc_slice(zeros_v, acc, sid):
  rows = ACC_R // 16
  base = sid * rows
  pltpu.sync_copy(zeros_v, acc.at[pl.ds(base, C)])
  pltpu.sync_copy(zeros_v, acc.at[pl.ds(base + C, C)])
  pltpu.sync_copy(zeros_v.at[pl.ds(0, rows - 2 * C)],
                  acc.at[pl.ds(base + 2 * C, rows - 2 * C)])


def _acc_to_out(acc, out_hbm, cid, sid):
  for o, sz in ((0, C), (C, C), (2 * C, OPT - 2 * C)):
    pltpu.sync_copy(acc.at[pl.ds(sid * OPT + o, sz)],
                    out_hbm.at[pl.ds(cid * HALF + sid * OPT + o, sz)])


def _make_deg_kernel(mesh):
  @functools.partial(
      pl.kernel,
      out_type=jax.ShapeDtypeStruct((NPAD, D), jnp.float32),
      mesh=mesh,
      scratch_types=[
          pltpu.VMEM((CPT, C), jnp.int32),
          pltpu.VMEM((1, C), jnp.int32),
          pltpu.VMEM((C, D), jnp.float32),
          pltpu.VMEM((C, D), jnp.float32),
          pltpu.VMEM_SHARED((ACC_R, D), jnp.float32),
      ],
  )
  def deg_kernel(dst_hbm, out_hbm, dst_v, lidx, ones_v, zeros_v, acc):
    cid = lax.axis_index("c")
    sid = lax.axis_index("s")
    base = cid * HALF
    pltpu.sync_copy(dst_hbm.at[pl.ds(sid * CPT, CPT)], dst_v)
    _fill_rows(ones_v, C, 1.0)
    _fill_rows(zeros_v, C, 0.0)
    _zero_acc_slice(zeros_v, acc, sid)
    plsc.subcore_barrier()

    def body(k, carry):
      _local_idx(dst_v, k, base, lidx)
      pltpu.sync_copy(ones_v, acc.at[lidx.at[0]], add=True)
      return carry

    lax.fori_loop(0, CPT, body, 0)
    plsc.subcore_barrier()
    _acc_to_out(acc, out_hbm, cid, sid)

  return deg_kernel


def _make_agg_kernel(mesh):
  @functools.partial(
      pl.kernel,
      out_type=jax.ShapeDtypeStruct((NPAD, D), jnp.float32),
      mesh=mesh,
      scratch_types=[
          pltpu.VMEM((PH, C), jnp.int32),
          pltpu.VMEM((PH, C), jnp.int32),
          pltpu.VMEM((1, C), jnp.int32),
          pltpu.VMEM((C, D), jnp.float32),
          pltpu.VMEM((C, D), jnp.float32),
          pltpu.VMEM((C, D), jnp.float32),
          pltpu.SemaphoreType.DMA,
          pltpu.SemaphoreType.DMA,
          pltpu.VMEM_SHARED((ACC_R, D), jnp.float32),
      ],
  )
  def agg_kernel(hp_hbm, src_hbm, dst_hbm, out_hbm,
                 src_v, dst_v, lidx, rows0, rows1, zeros_v, g0, g1, acc):
    cid = lax.axis_index("c")
    sid = lax.axis_index("s")
    base = cid * HALF
    _fill_rows(zeros_v, C, 0.0)
    _zero_acc_slice(zeros_v, acc, sid)
    plsc.subcore_barrier()

    for p in range(CPT // PH):
      pltpu.sync_copy(src_hbm.at[pl.ds(sid * CPT + p * PH, PH)], src_v)
      pltpu.sync_copy(dst_hbm.at[pl.ds(sid * CPT + p * PH, PH)], dst_v)
      pltpu.make_async_copy(hp_hbm.at[src_v.at[0]], rows0, g0).start()

      def body(t, carry):
        k0 = 2 * t
        k1 = k0 + 1
        pltpu.make_async_copy(hp_hbm.at[src_v.at[k0]], rows0, g0).wait()
        pltpu.make_async_copy(hp_hbm.at[src_v.at[k1]], rows1, g1).start()
        _local_idx(dst_v, k0, base, lidx)
        pltpu.sync_copy(rows0, acc.at[lidx.at[0]], add=True)
        pltpu.make_async_copy(hp_hbm.at[src_v.at[k1]], rows1, g1).wait()

        @pl.when(t < PH // 2 - 1)
        def _():
          pltpu.make_async_copy(hp_hbm.at[src_v.at[k0 + 2]], rows0, g0).start()

        _local_idx(dst_v, k1, base, lidx)
        pltpu.sync_copy(rows1, acc.at[lidx.at[0]], add=True)
        return carry

      lax.fori_loop(0, PH // 2, body, 0)
    plsc.subcore_barrier()
    _acc_to_out(acc, out_hbm, cid, sid)

  return agg_kernel


def _make_pool_kernel(mesh):
  @functools.partial(
      pl.kernel,
      out_type=jax.ShapeDtypeStruct((2, POOL_ROWS // 2, D), jnp.float32),
      mesh=mesh,
      scratch_types=[
          pltpu.VMEM((1, C), jnp.int32),
          pltpu.VMEM((1, C), jnp.int32),
          pltpu.VMEM((C, D), jnp.float32),
          pltpu.VMEM((C, D), jnp.float32),
          pltpu.VMEM((16, D), jnp.float32),
          pltpu.VMEM_SHARED((POOL_ROWS, D), jnp.float32),
      ],
  )
  def pool_kernel(h3_hbm, bat_hbm, bat64_hbm, out_hbm,
                  bidx, b64idx, rows_v, ones_v, zeros_v, acc):
    cid = lax.axis_index("c")
    sid = lax.axis_index("s")
    w = sid * 2 + cid
    _fill_rows(ones_v, C, 1.0)
    _fill_rows(zeros_v, 16, 0.0)
    pltpu.sync_copy(zeros_v, acc.at[pl.ds(sid * 16, 16)])
    plsc.subcore_barrier()
    for j in range(3):
      c = w + 32 * j

      @pl.when(c < PCHUNKS)
      def _():
        pltpu.sync_copy(bat_hbm.at[pl.ds(c, 1)], bidx)
        pltpu.sync_copy(bat64_hbm.at[pl.ds(c, 1)], b64idx)
        pltpu.sync_copy(h3_hbm.at[pl.ds(c * C, C)], rows_v)
        pltpu.sync_copy(rows_v, acc.at[bidx.at[0]], add=True)
        pltpu.sync_copy(ones_v, acc.at[b64idx.at[0]], add=True)

    plsc.subcore_barrier()
    pltpu.sync_copy(acc.at[pl.ds(sid * 8, 8)],
                    out_hbm.at[cid, pl.ds(sid * 8, 8)])

  return pool_kernel


def _tc_pre_body(x_ref, w_ref, degp_ref, hp_ref, dinv_ref):
  deg = degp_ref[...] + 1.0
  dinv = lax.rsqrt(deg)
  z = jnp.dot(x_ref[...], w_ref[...], preferred_element_type=jnp.float32)
  hp_ref[...] = dinv * z
  dinv_ref[...] = dinv[:, 0:1]


def _tc_mid_body(p_ref, hp_ref, dinv_ref, b_ref, w_ref, out_ref):
  agg = p_ref[...] + hp_ref[...]
  h = jnp.maximum(dinv_ref[...] * agg + b_ref[...], 0.0)
  out_ref[...] = dinv_ref[...] * jnp.dot(
      h, w_ref[...], preferred_element_type=jnp.float32)


def _tc_post_body(p_ref, hp_ref, dinv_ref, b_ref, out_ref):
  agg = p_ref[...] + hp_ref[...]
  out_ref[...] = jnp.maximum(dinv_ref[...] * agg + b_ref[...], 0.0)


def _tc_head_body(pp_ref, fw1_ref, fb1_ref, fw2_ref, fb2_ref, out_ref):
  p = pp_ref[0] + pp_ref[1]
  s = p[0:NG, :]
  cnt = p[NG:2 * NG, 0:1]
  g = s / jnp.maximum(cnt, 1.0)
  r = jnp.maximum(
      jnp.dot(g, fw1_ref[...], preferred_element_type=jnp.float32)
      + fb1_ref[...], 0.0)
  out_ref[...] = jnp.dot(
      r, fw2_ref[...], preferred_element_type=jnp.float32) + fb2_ref[...]


_R = 1024
_G = NPAD // _R


def _row_spec(bl=_R):
  return pl.BlockSpec((bl, D), lambda i: (i, 0))


def _tc_pre(xp, W1, degp):
  return pl.pallas_call(
      _tc_pre_body,
      grid=(_G,),
      in_specs=[
          _row_spec(),
          pl.BlockSpec((D, D), lambda i: (0, 0)),
          _row_spec(),
      ],
      out_specs=[
          _row_spec(),
          pl.BlockSpec((_R, 1), lambda i: (i, 0)),
      ],
      out_shape=[
          jax.ShapeDtypeStruct((NPAD, D), jnp.float32),
          jax.ShapeDtypeStruct((NPAD, 1), jnp.float32),
      ],
  )(xp, W1, degp)


def _tc_mid(parts, hp, dinv, b, W):
  return pl.pallas_call(
      _tc_mid_body,
      grid=(_G,),
      in_specs=[
          _row_spec(),
          _row_spec(),
          pl.BlockSpec((_R, 1), lambda i: (i, 0)),
          pl.BlockSpec((1, D), lambda i: (0, 0)),
          pl.BlockSpec((D, D), lambda i: (0, 0)),
      ],
      out_specs=_row_spec(),
      out_shape=jax.ShapeDtypeStruct((NPAD, D), jnp.float32),
  )(parts, hp, dinv, b, W)


def _tc_post(parts, hp, dinv, b):
  return pl.pallas_call(
      _tc_post_body,
      grid=(_G,),
      in_specs=[
          _row_spec(),
          _row_spec(),
          pl.BlockSpec((_R, 1), lambda i: (i, 0)),
          pl.BlockSpec((1, D), lambda i: (0, 0)),
      ],
      out_specs=_row_spec(),
      out_shape=jax.ShapeDtypeStruct((NPAD, D), jnp.float32),
  )(parts, hp, dinv, b)


def _tc_head(pool, fW1, fb1, fW2, fb2):
  return pl.pallas_call(
      _tc_head_body,
      out_shape=jax.ShapeDtypeStruct((NG, NCLS), jnp.float32),
  )(pool, fW1, fb1, fW2, fb2)


def kernel(x, edge_index, batch, W1, b1, W2, b2, W3, b3, fW1, fb1, fW2, fb2):
  mesh = plsc.VectorSubcoreMesh(core_axis_name="c", subcore_axis_name="s")
  deg_k = _make_deg_kernel(mesh)
  agg_k = _make_agg_kernel(mesh)
  pool_k = _make_pool_kernel(mesh)

  src = edge_index[0].astype(jnp.int32)
  dst = edge_index[1].astype(jnp.int32)
  srcp = jnp.concatenate(
      [src, jnp.full((EPAD - E,), DUMMY, jnp.int32)]).reshape(ECHUNKS, C)
  dstp = jnp.concatenate(
      [dst, jnp.full((EPAD - E,), NPAD, jnp.int32)]).reshape(ECHUNKS, C)
  xp = jnp.zeros((NPAD, D), jnp.float32).at[:N].set(x)
  batp = jnp.concatenate(
      [batch.astype(jnp.int32), jnp.full((NPAD - N,), TRASH, jnp.int32)]
  ).reshape(PCHUNKS, C)
  batp64 = batp + NG

  degp = deg_k(dstp)
  hp1, dinv = _tc_pre(xp, W1, degp)
  parts1 = agg_k(hp1, srcp, dstp)
  hp2 = _tc_mid(parts1, hp1, dinv, b1.reshape(1, D), W2)
  parts2 = agg_k(hp2, srcp, dstp)
  hp3 = _tc_mid(parts2, hp2, dinv, b2.reshape(1, D), W3)
  parts3 = agg_k(hp3, srcp, dstp)
  h3 = _tc_post(parts3, hp3, dinv, b3.reshape(1, D))
  pool = pool_k(h3, batp, batp64)
  return _tc_head(pool, fW1, fb1.reshape(1, D), fW2, fb2.reshape(1, NCLS))


# speedup vs baseline: 5.0377x; 1.0418x over previous
import functools

import jax
import jax.numpy as jnp
from jax import lax
from jax.experimental import pallas as pl
from jax.experimental.pallas import tpu as pltpu
from jax.experimental.pallas import tpu_sc as plsc

N = 10000
E = 320000
D = 128
NCLS = 16
NG = 64

NPAD = 10240
DUMMY = N
C = 128
ECHUNKS = 2560
EPAD = ECHUNKS * C
CPT = ECHUNKS // 16
PH = 40
HALF = NPAD // 2
ACC_R = HALF + 256
ATRASH = HALF
OPT = HALF // 16
PCHUNKS = NPAD // C
POOL_ROWS = 256
TRASH = 128


def _fill_rows(ref, nrows, value):
  vec = jnp.full((16,), value, jnp.float32)

  def body(i, carry):
    for cc in range(8):
      ref[i, pl.ds(cc * 16, 16)] = vec
    return carry

  lax.fori_loop(0, nrows, body, 0)


def _local_idx(dst_v, k, base, lidx, row=0):
  lane = lax.iota(jnp.int32, 16)
  for cc in range(8):
    d = dst_v[k, pl.ds(cc * 16, 16)]
    ld = d - base
    ok = (ld >= 0) & (ld < HALF)
    lidx[row, pl.ds(cc * 16, 16)] = jnp.where(ok, ld, ATRASH + cc * 16 + lane)


def _zero_acc_slice(zeros_v, acc, sid):
  rows = ACC_R // 16
  base = sid * rows
  pltpu.sync_copy(zeros_v, acc.at[pl.ds(base, C)])
  pltpu.sync_copy(zeros_v, acc.at[pl.ds(base + C, C)])
  pltpu.sync_copy(zeros_v.at[pl.ds(0, rows - 2 * C)],
                  acc.at[pl.ds(base + 2 * C, rows - 2 * C)])


def _acc_to_out(acc, out_hbm, cid, sid):
  for o, sz in ((0, C), (C, C), (2 * C, OPT - 2 * C)):
    pltpu.sync_copy(acc.at[pl.ds(sid * OPT + o, sz)],
                    out_hbm.at[pl.ds(cid * HALF + sid * OPT + o, sz)])


def _make_deg_kernel(mesh):
  @functools.partial(
      pl.kernel,
      out_type=jax.ShapeDtypeStruct((NPAD, D), jnp.float32),
      mesh=mesh,
      scratch_types=[
          pltpu.VMEM((CPT, C), jnp.int32),
          pltpu.VMEM((1, C), jnp.int32),
          pltpu.VMEM((C, D), jnp.float32),
          pltpu.VMEM((C, D), jnp.float32),
          pltpu.VMEM_SHARED((ACC_R, D), jnp.float32),
      ],
  )
  def deg_kernel(dst_hbm, out_hbm, dst_v, lidx, ones_v, zeros_v, acc):
    cid = lax.axis_index("c")
    sid = lax.axis_index("s")
    base = cid * HALF
    pltpu.sync_copy(dst_hbm.at[pl.ds(sid * CPT, CPT)], dst_v)
    _fill_rows(ones_v, C, 1.0)
    _fill_rows(zeros_v, C, 0.0)
    _zero_acc_slice(zeros_v, acc, sid)
    plsc.subcore_barrier()

    def body(k, carry):
      _local_idx(dst_v, k, base, lidx)
      pltpu.sync_copy(ones_v, acc.at[lidx.at[0]], add=True)
      return carry

    lax.fori_loop(0, CPT, body, 0)
    plsc.subcore_barrier()
    _acc_to_out(acc, out_hbm, cid, sid)

  return deg_kernel


def _make_agg_kernel(mesh):
  @functools.partial(
      pl.kernel,
      out_type=jax.ShapeDtypeStruct((NPAD, D), jnp.float32),
      mesh=mesh,
      scratch_types=[
          pltpu.VMEM((PH, C), jnp.int32),
          pltpu.VMEM((PH, C), jnp.int32),
          pltpu.VMEM((4, C), jnp.int32),
          [pltpu.VMEM((C, D), jnp.float32) for _ in range(4)],
          [pltpu.SemaphoreType.DMA for _ in range(4)],
          [pltpu.SemaphoreType.DMA for _ in range(4)],
          pltpu.VMEM_SHARED((ACC_R, D), jnp.float32),
      ],
  )
  def agg_kernel(hp_hbm, src_hbm, dst_hbm, out_hbm,
                 src_v, dst_v, lidx, rows, gsem, ssem, acc):
    cid = lax.axis_index("c")
    sid = lax.axis_index("s")
    base = cid * HALF
    _fill_rows(rows[0], C, 0.0)
    _zero_acc_slice(rows[0], acc, sid)
    plsc.subcore_barrier()

    def gather(k, b):
      return pltpu.make_async_copy(hp_hbm.at[src_v.at[k]], rows[b], gsem[b])

    def scatter(b):
      return pltpu.make_async_copy(rows[b], acc.at[lidx.at[b]], ssem[b])

    for p in range(CPT // PH):
      pltpu.sync_copy(src_hbm.at[pl.ds(sid * CPT + p * PH, PH)], src_v)
      pltpu.sync_copy(dst_hbm.at[pl.ds(sid * CPT + p * PH, PH)], dst_v)
      gather(0, 0).start()
      gather(1, 1).start()

      def body(t, carry):
        for i in range(4):
          k = 4 * t + i
          b = i
          bn = (i + 2) % 4
          if i < 2:
            @pl.when(t > 0)
            def _():
              scatter(bn).wait()
            gather(k + 2, bn).start()
          else:
            scatter(bn).wait()

            @pl.when(t < PH // 4 - 1)
            def _():
              gather(k + 2, bn).start()

          gather(k, b).wait()
          _local_idx(dst_v, k, base, lidx, row=b)
          scatter(b).start(add=True)
        return carry

      lax.fori_loop(0, PH // 4, body, 0)
      scatter((PH - 2) % 4).wait()
      scatter((PH - 1) % 4).wait()
    plsc.subcore_barrier()
    _acc_to_out(acc, out_hbm, cid, sid)

  return agg_kernel


def _make_pool_kernel(mesh):
  @functools.partial(
      pl.kernel,
      out_type=jax.ShapeDtypeStruct((2, POOL_ROWS // 2, D), jnp.float32),
      mesh=mesh,
      scratch_types=[
          pltpu.VMEM((1, C), jnp.int32),
          pltpu.VMEM((1, C), jnp.int32),
          pltpu.VMEM((C, D), jnp.float32),
          pltpu.VMEM((C, D), jnp.float32),
          pltpu.VMEM((16, D), jnp.float32),
          pltpu.VMEM_SHARED((POOL_ROWS, D), jnp.float32),
      ],
  )
  def pool_kernel(h3_hbm, bat_hbm, bat64_hbm, out_hbm,
                  bidx, b64idx, rows_v, ones_v, zeros_v, acc):
    cid = lax.axis_index("c")
    sid = lax.axis_index("s")
    w = sid * 2 + cid
    _fill_rows(ones_v, C, 1.0)
    _fill_rows(zeros_v, 16, 0.0)
    pltpu.sync_copy(zeros_v, acc.at[pl.ds(sid * 16, 16)])
    plsc.subcore_barrier()
    for j in range(3):
      c = w + 32 * j

      @pl.when(c < PCHUNKS)
      def _():
        pltpu.sync_copy(bat_hbm.at[pl.ds(c, 1)], bidx)
        pltpu.sync_copy(bat64_hbm.at[pl.ds(c, 1)], b64idx)
        pltpu.sync_copy(h3_hbm.at[pl.ds(c * C, C)], rows_v)
        pltpu.sync_copy(rows_v, acc.at[bidx.at[0]], add=True)
        pltpu.sync_copy(ones_v, acc.at[b64idx.at[0]], add=True)

    plsc.subcore_barrier()
    pltpu.sync_copy(acc.at[pl.ds(sid * 8, 8)],
                    out_hbm.at[cid, pl.ds(sid * 8, 8)])

  return pool_kernel


def _tc_pre_body(x_ref, w_ref, degp_ref, hp_ref, dinv_ref):
  deg = degp_ref[...] + 1.0
  dinv = lax.rsqrt(deg)
  z = jnp.dot(x_ref[...], w_ref[...], preferred_element_type=jnp.float32)
  hp_ref[...] = dinv * z
  dinv_ref[...] = dinv[:, 0:1]


def _tc_mid_body(p_ref, hp_ref, dinv_ref, b_ref, w_ref, out_ref):
  agg = p_ref[...] + hp_ref[...]
  h = jnp.maximum(dinv_ref[...] * agg + b_ref[...], 0.0)
  out_ref[...] = dinv_ref[...] * jnp.dot(
      h, w_ref[...], preferred_element_type=jnp.float32)


def _tc_post_body(p_ref, hp_ref, dinv_ref, b_ref, out_ref):
  agg = p_ref[...] + hp_ref[...]
  out_ref[...] = jnp.maximum(dinv_ref[...] * agg + b_ref[...], 0.0)


def _tc_head_body(pp_ref, fw1_ref, fb1_ref, fw2_ref, fb2_ref, out_ref):
  p = pp_ref[0] + pp_ref[1]
  s = p[0:NG, :]
  cnt = p[NG:2 * NG, 0:1]
  g = s / jnp.maximum(cnt, 1.0)
  r = jnp.maximum(
      jnp.dot(g, fw1_ref[...], preferred_element_type=jnp.float32)
      + fb1_ref[...], 0.0)
  out_ref[...] = jnp.dot(
      r, fw2_ref[...], preferred_element_type=jnp.float32) + fb2_ref[...]


_R = 1024
_G = NPAD // _R


def _row_spec(bl=_R):
  return pl.BlockSpec((bl, D), lambda i: (i, 0))


def _tc_pre(xp, W1, degp):
  return pl.pallas_call(
      _tc_pre_body,
      grid=(_G,),
      in_specs=[
          _row_spec(),
          pl.BlockSpec((D, D), lambda i: (0, 0)),
          _row_spec(),
      ],
      out_specs=[
          _row_spec(),
          pl.BlockSpec((_R, 1), lambda i: (i, 0)),
      ],
      out_shape=[
          jax.ShapeDtypeStruct((NPAD, D), jnp.float32),
          jax.ShapeDtypeStruct((NPAD, 1), jnp.float32),
      ],
  )(xp, W1, degp)


def _tc_mid(parts, hp, dinv, b, W):
  return pl.pallas_call(
      _tc_mid_body,
      grid=(_G,),
      in_specs=[
          _row_spec(),
          _row_spec(),
          pl.BlockSpec((_R, 1), lambda i: (i, 0)),
          pl.BlockSpec((1, D), lambda i: (0, 0)),
          pl.BlockSpec((D, D), lambda i: (0, 0)),
      ],
      out_specs=_row_spec(),
      out_shape=jax.ShapeDtypeStruct((NPAD, D), jnp.float32),
  )(parts, hp, dinv, b, W)


def _tc_post(parts, hp, dinv, b):
  return pl.pallas_call(
      _tc_post_body,
      grid=(_G,),
      in_specs=[
          _row_spec(),
          _row_spec(),
          pl.BlockSpec((_R, 1), lambda i: (i, 0)),
          pl.BlockSpec((1, D), lambda i: (0, 0)),
      ],
      out_specs=_row_spec(),
      out_shape=jax.ShapeDtypeStruct((NPAD, D), jnp.float32),
  )(parts, hp, dinv, b)


def _tc_head(pool, fW1, fb1, fW2, fb2):
  return pl.pallas_call(
      _tc_head_body,
      out_shape=jax.ShapeDtypeStruct((NG, NCLS), jnp.float32),
  )(pool, fW1, fb1, fW2, fb2)


def kernel(x, edge_index, batch, W1, b1, W2, b2, W3, b3, fW1, fb1, fW2, fb2):
  mesh = plsc.VectorSubcoreMesh(core_axis_name="c", subcore_axis_name="s")
  deg_k = _make_deg_kernel(mesh)
  agg_k = _make_agg_kernel(mesh)
  pool_k = _make_pool_kernel(mesh)

  src = edge_index[0].astype(jnp.int32)
  dst = edge_index[1].astype(jnp.int32)
  srcp = jnp.concatenate(
      [src, jnp.full((EPAD - E,), DUMMY, jnp.int32)]).reshape(ECHUNKS, C)
  dstp = jnp.concatenate(
      [dst, jnp.full((EPAD - E,), NPAD, jnp.int32)]).reshape(ECHUNKS, C)
  xp = jnp.zeros((NPAD, D), jnp.float32).at[:N].set(x)
  batp = jnp.concatenate(
      [batch.astype(jnp.int32), jnp.full((NPAD - N,), TRASH, jnp.int32)]
  ).reshape(PCHUNKS, C)
  batp64 = batp + NG

  degp = deg_k(dstp)
  hp1, dinv = _tc_pre(xp, W1, degp)
  parts1 = agg_k(hp1, srcp, dstp)
  hp2 = _tc_mid(parts1, hp1, dinv, b1.reshape(1, D), W2)
  parts2 = agg_k(hp2, srcp, dstp)
  hp3 = _tc_mid(parts2, hp2, dinv, b2.reshape(1, D), W3)
  parts3 = agg_k(hp3, srcp, dstp)
  h3 = _tc_post(parts3, hp3, dinv, b3.reshape(1, D))
  pool = pool_k(h3, batp, batp64)
  return _tc_head(pool, fW1, fb1.reshape(1, D), fW2, fb2.reshape(1, NCLS))


# speedup vs baseline: 5.3375x; 1.0595x over previous
import functools

import jax
import jax.numpy as jnp
from jax import lax
from jax.experimental import pallas as pl
from jax.experimental.pallas import tpu as pltpu
from jax.experimental.pallas import tpu_sc as plsc

N = 10000
E = 320000
D = 128
NCLS = 16
NG = 64

NPAD = 10240
DUMMY = N
C = 128
ECHUNKS = 2560
EPAD = ECHUNKS * C
CPW = 80
HALF = NPAD // 2
ACC_R = HALF + 256
ATRASH = HALF
OPT = HALF // 16
RCAP = 10496
RCH = RCAP // C
PCHUNKS = NPAD // C
POOL_ROWS = 256
TRASH = 128


def _fill_rows(ref, nrows, value):
  vec = jnp.full((16,), value, jnp.float32)

  def body(i, carry):
    for cc in range(8):
      ref[i, pl.ds(cc * 16, 16)] = vec
    return carry

  lax.fori_loop(0, nrows, body, 0)


def _zero_acc_slice(zeros_v, acc, sid):
  rows = ACC_R // 16
  base = sid * rows
  pltpu.sync_copy(zeros_v, acc.at[pl.ds(base, C)])
  pltpu.sync_copy(zeros_v, acc.at[pl.ds(base + C, C)])
  pltpu.sync_copy(zeros_v.at[pl.ds(0, rows - 2 * C)],
                  acc.at[pl.ds(base + 2 * C, rows - 2 * C)])


def _acc_to_out(acc, out_hbm, cid, sid):
  for o, sz in ((0, C), (C, C), (2 * C, OPT - 2 * C)):
    pltpu.sync_copy(acc.at[pl.ds(sid * OPT + o, sz)],
                    out_hbm.at[pl.ds(cid * HALF + sid * OPT + o, sz)])


def _popcount(mask):
  return jnp.max(plsc.all_reduce_population_count(mask))


def _make_part_kernel(mesh):
  @functools.partial(
      pl.kernel,
      out_type=[
          jax.ShapeDtypeStruct((64 * RCAP,), jnp.int32),
          jax.ShapeDtypeStruct((64 * RCAP,), jnp.int32),
          jax.ShapeDtypeStruct((512,), jnp.int32),
      ],
      mesh=mesh,
      scratch_types=[
          pltpu.VMEM((CPW, C), jnp.int32),
          pltpu.VMEM((CPW, C), jnp.int32),
          [pltpu.VMEM((RCAP,), jnp.int32) for _ in range(2)],
          [pltpu.VMEM((RCAP,), jnp.int32) for _ in range(2)],
          pltpu.VMEM((16,), jnp.int32),
          pltpu.SemaphoreType.DMA,
      ],
      compiler_params=pltpu.CompilerParams(needs_layout_passes=False),
  )
  def part_kernel(src_hbm, dst_hbm, psrc_hbm, pdst_hbm, cnt_hbm,
                  src_v, dst_v, outs, outd, cbuf, sem):
    cid = lax.axis_index("c")
    sid = lax.axis_index("s")
    w = sid * 2 + cid
    pltpu.sync_copy(src_hbm.at[pl.ds(w * CPW, CPW)], src_v)
    pltpu.sync_copy(dst_hbm.at[pl.ds(w * CPW, CPW)], dst_v)

    def body(k, carry):
      c0, c1 = carry
      for cc in range(8):
        s = src_v[k, pl.ds(cc * 16, 16)]
        d = dst_v[k, pl.ds(cc * 16, 16)]
        m0 = d < HALF
        m1 = (d >= HALF) & (d < NPAD)
        cum0 = plsc.cumsum(m0.astype(jnp.int32))
        cum1 = plsc.cumsum(m1.astype(jnp.int32))
        p0 = c0 + cum0 - 1
        p1 = c1 + cum1 - 1
        plsc.store_scatter(outs[0], [p0], s, mask=m0)
        plsc.store_scatter(outd[0], [p0], d, mask=m0)
        plsc.store_scatter(outs[1], [p1], s, mask=m1)
        plsc.store_scatter(outd[1], [p1], d - HALF, mask=m1)
        c0 = c0 + jnp.max(cum0)
        c1 = c1 + jnp.max(cum1)
      return (c0, c1)

    c0, c1 = lax.fori_loop(0, CPW, body, (jnp.int32(0), jnp.int32(0)))

    lane = lax.iota(jnp.int32, 16)
    dummy_vec = jnp.full((16,), DUMMY, jnp.int32)
    for h, cnt in ((0, c0), (1, c1)):
      tgt = ((cnt + 511) // 512) * 512

      def pad_body(i, carry):
        off = cnt + i * 16

        @pl.when(off < tgt)
        def _():
          plsc.store_scatter(outs[h], [off + lane], dummy_vec)
          plsc.store_scatter(outd[h], [off + lane],
                             ATRASH + ((i * 16 + lane) & 127))

        return carry

      lax.fori_loop(0, 32, pad_body, 0)

      cbuf[...] = jnp.broadcast_to(tgt // 512, (16,)).astype(jnp.int32)
      pltpu.sync_copy(cbuf.at[pl.ds(0, 8)],
                      cnt_hbm.at[pl.ds(w * 16 + h * 8, 8)])
      pltpu.sync_copy(outs[h], psrc_hbm.at[pl.ds((h * 32 + w) * RCAP, RCAP)])
      pltpu.sync_copy(outd[h], pdst_hbm.at[pl.ds((h * 32 + w) * RCAP, RCAP)])

  return part_kernel


def _make_deg_kernel(mesh):
  @functools.partial(
      pl.kernel,
      out_type=jax.ShapeDtypeStruct((NPAD, D), jnp.float32),
      mesh=mesh,
      scratch_types=[
          pltpu.VMEM((RCAP,), jnp.int32),
          pltpu.VMEM((16,), jnp.int32),
          pltpu.VMEM((C, D), jnp.float32),
          [pltpu.SemaphoreType.DMA for _ in range(2)],
          pltpu.VMEM_SHARED((ACC_R, D), jnp.float32),
      ],
  )
  def deg_kernel(pdst_hbm, cnt_hbm, out_hbm, dst_v, cnt_v, buf, sems, acc):
    cid = lax.axis_index("c")
    sid = lax.axis_index("s")
    _fill_rows(buf, C, 0.0)
    _zero_acc_slice(buf, acc, sid)
    plsc.subcore_barrier()
    _fill_rows(buf, C, 1.0)

    def scatter(k, b):
      return pltpu.make_async_copy(
          buf, acc.at[dst_v.at[pl.ds(k * C, C)]], sems[b])

    for rr in range(2):
      w = 2 * sid + rr
      ridx = cid * 32 + w
      pltpu.sync_copy(pdst_hbm.at[pl.ds(ridx * RCAP, RCAP)], dst_v)
      pltpu.sync_copy(cnt_hbm.at[pl.ds(w * 16 + cid * 8, 8)],
                      cnt_v.at[pl.ds(0, 8)])
      t4 = cnt_v[...][0]

      @pl.when(t4 > 0)
      def _():
        def body(t, carry):
          for i in range(2):
            k = 2 * t + i

            @pl.when(t > 0)
            def _():
              scatter(k - 2, i).wait()

            scatter(k, i).start(add=True)
          return carry

        lax.fori_loop(0, 2 * t4, body, 0)
        scatter(4 * t4 - 2, 0).wait()
        scatter(4 * t4 - 1, 1).wait()

    plsc.subcore_barrier()
    _acc_to_out(acc, out_hbm, cid, sid)

  return deg_kernel


def _make_agg_kernel(mesh):
  @functools.partial(
      pl.kernel,
      out_type=jax.ShapeDtypeStruct((NPAD, D), jnp.float32),
      mesh=mesh,
      scratch_types=[
          pltpu.VMEM((RCAP,), jnp.int32),
          pltpu.VMEM((RCAP,), jnp.int32),
          pltpu.VMEM((16,), jnp.int32),
          [pltpu.VMEM((C, D), jnp.float32) for _ in range(4)],
          [pltpu.SemaphoreType.DMA for _ in range(4)],
          [pltpu.SemaphoreType.DMA for _ in range(4)],
          pltpu.VMEM_SHARED((ACC_R, D), jnp.float32),
      ],
  )
  def agg_kernel(hp_hbm, psrc_hbm, pdst_hbm, cnt_hbm, out_hbm,
                 src_v, dst_v, cnt_v, rows, gsem, ssem, acc):
    cid = lax.axis_index("c")
    sid = lax.axis_index("s")
    _fill_rows(rows[0], C, 0.0)
    _zero_acc_slice(rows[0], acc, sid)
    plsc.subcore_barrier()

    def gather(k, b):
      return pltpu.make_async_copy(
          hp_hbm.at[src_v.at[pl.ds(k * C, C)]], rows[b], gsem[b])

    def scatter(k, b):
      return pltpu.make_async_copy(
          rows[b], acc.at[dst_v.at[pl.ds(k * C, C)]], ssem[b])

    for rr in range(2):
      w = 2 * sid + rr
      ridx = cid * 32 + w
      pltpu.sync_copy(psrc_hbm.at[pl.ds(ridx * RCAP, RCAP)], src_v)
      pltpu.sync_copy(pdst_hbm.at[pl.ds(ridx * RCAP, RCAP)], dst_v)
      pltpu.sync_copy(cnt_hbm.at[pl.ds(w * 16 + cid * 8, 8)],
                      cnt_v.at[pl.ds(0, 8)])
      t4 = cnt_v[...][0]

      @pl.when(t4 > 0)
      def _():
        gather(0, 0).start()
        gather(1, 1).start()

        def body(t, carry):
          for i in range(4):
            k = 4 * t + i
            b = i
            bn = (i + 2) % 4
            if i < 2:
              @pl.when(t > 0)
              def _():
                scatter(k - 2, bn).wait()

              gather(k + 2, bn).start()
            else:
              scatter(k - 2, bn).wait()

              @pl.when(t < t4 - 1)
              def _():
                gather(k + 2, bn).start()

            gather(k, b).wait()
            scatter(k, b).start(add=True)
          return carry

        lax.fori_loop(0, t4, body, 0)
        scatter(4 * t4 - 2, 2).wait()
        scatter(4 * t4 - 1, 3).wait()

    plsc.subcore_barrier()
    _acc_to_out(acc, out_hbm, cid, sid)

  return agg_kernel


def _make_pool_kernel(mesh):
  @functools.partial(
      pl.kernel,
      out_type=jax.ShapeDtypeStruct((2, POOL_ROWS // 2, D), jnp.float32),
      mesh=mesh,
      scratch_types=[
          pltpu.VMEM((1, C), jnp.int32),
          pltpu.VMEM((1, C), jnp.int32),
          pltpu.VMEM((C, D), jnp.float32),
          pltpu.VMEM((C, D), jnp.float32),
          pltpu.VMEM((16, D), jnp.float32),
          pltpu.VMEM_SHARED((POOL_ROWS, D), jnp.float32),
      ],
  )
  def pool_kernel(h3_hbm, bat_hbm, bat64_hbm, out_hbm,
                  bidx, b64idx, rows_v, ones_v, zeros_v, acc):
    cid = lax.axis_index("c")
    sid = lax.axis_index("s")
    w = sid * 2 + cid
    _fill_rows(ones_v, C, 1.0)
    _fill_rows(zeros_v, 16, 0.0)
    pltpu.sync_copy(zeros_v, acc.at[pl.ds(sid * 16, 16)])
    plsc.subcore_barrier()
    for j in range(3):
      c = w + 32 * j

      @pl.when(c < PCHUNKS)
      def _():
        pltpu.sync_copy(bat_hbm.at[pl.ds(c, 1)], bidx)
        pltpu.sync_copy(bat64_hbm.at[pl.ds(c, 1)], b64idx)
        pltpu.sync_copy(h3_hbm.at[pl.ds(c * C, C)], rows_v)
        pltpu.sync_copy(rows_v, acc.at[bidx.at[0]], add=True)
        pltpu.sync_copy(ones_v, acc.at[b64idx.at[0]], add=True)

    plsc.subcore_barrier()
    pltpu.sync_copy(acc.at[pl.ds(sid * 8, 8)],
                    out_hbm.at[cid, pl.ds(sid * 8, 8)])

  return pool_kernel


def _tc_pre_body(x_ref, w_ref, degp_ref, hp_ref, dinv_ref):
  deg = degp_ref[...] + 1.0
  dinv = lax.rsqrt(deg)
  z = jnp.dot(x_ref[...], w_ref[...], preferred_element_type=jnp.float32)
  hp_ref[...] = dinv * z
  dinv_ref[...] = dinv[:, 0:1]


def _tc_mid_body(p_ref, hp_ref, dinv_ref, b_ref, w_ref, out_ref):
  agg = p_ref[...] + hp_ref[...]
  h = jnp.maximum(dinv_ref[...] * agg + b_ref[...], 0.0)
  out_ref[...] = dinv_ref[...] * jnp.dot(
      h, w_ref[...], preferred_element_type=jnp.float32)


def _tc_post_body(p_ref, hp_ref, dinv_ref, b_ref, out_ref):
  agg = p_ref[...] + hp_ref[...]
  out_ref[...] = jnp.maximum(dinv_ref[...] * agg + b_ref[...], 0.0)


def _tc_head_body(pp_ref, fw1_ref, fb1_ref, fw2_ref, fb2_ref, out_ref):
  p = pp_ref[0] + pp_ref[1]
  s = p[0:NG, :]
  cnt = p[NG:2 * NG, 0:1]
  g = s / jnp.maximum(cnt, 1.0)
  r = jnp.maximum(
      jnp.dot(g, fw1_ref[...], preferred_element_type=jnp.float32)
      + fb1_ref[...], 0.0)
  out_ref[...] = jnp.dot(
      r, fw2_ref[...], preferred_element_type=jnp.float32) + fb2_ref[...]


_R = 1024
_G = NPAD // _R


def _row_spec(bl=_R):
  return pl.BlockSpec((bl, D), lambda i: (i, 0))


def _tc_pre(xp, W1, degp):
  return pl.pallas_call(
      _tc_pre_body,
      grid=(_G,),
      in_specs=[
          _row_spec(),
          pl.BlockSpec((D, D), lambda i: (0, 0)),
          _row_spec(),
      ],
      out_specs=[
          _row_spec(),
          pl.BlockSpec((_R, 1), lambda i: (i, 0)),
      ],
      out_shape=[
          jax.ShapeDtypeStruct((NPAD, D), jnp.float32),
          jax.ShapeDtypeStruct((NPAD, 1), jnp.float32),
      ],
  )(xp, W1, degp)


def _tc_mid(parts, hp, dinv, b, W):
  return pl.pallas_call(
      _tc_mid_body,
      grid=(_G,),
      in_specs=[
          _row_spec(),
          _row_spec(),
          pl.BlockSpec((_R, 1), lambda i: (i, 0)),
          pl.BlockSpec((1, D), lambda i: (0, 0)),
          pl.BlockSpec((D, D), lambda i: (0, 0)),
      ],
      out_specs=_row_spec(),
      out_shape=jax.ShapeDtypeStruct((NPAD, D), jnp.float32),
  )(parts, hp, dinv, b, W)


def _tc_post(parts, hp, dinv, b):
  return pl.pallas_call(
      _tc_post_body,
      grid=(_G,),
      in_specs=[
          _row_spec(),
          _row_spec(),
          pl.BlockSpec((_R, 1), lambda i: (i, 0)),
          pl.BlockSpec((1, D), lambda i: (0, 0)),
      ],
      out_specs=_row_spec(),
      out_shape=jax.ShapeDtypeStruct((NPAD, D), jnp.float32),
  )(parts, hp, dinv, b)


def _tc_head(pool, fW1, fb1, fW2, fb2):
  return pl.pallas_call(
      _tc_head_body,
      out_shape=jax.ShapeDtypeStruct((NG, NCLS), jnp.float32),
  )(pool, fW1, fb1, fW2, fb2)


def kernel(x, edge_index, batch, W1, b1, W2, b2, W3, b3, fW1, fb1, fW2, fb2):
  mesh = plsc.VectorSubcoreMesh(core_axis_name="c", subcore_axis_name="s")
  part_k = _make_part_kernel(mesh)
  deg_k = _make_deg_kernel(mesh)
  agg_k = _make_agg_kernel(mesh)
  pool_k = _make_pool_kernel(mesh)

  src = edge_index[0].astype(jnp.int32)
  dst = edge_index[1].astype(jnp.int32)
  srcp = jnp.concatenate(
      [src, jnp.full((EPAD - E,), DUMMY, jnp.int32)]).reshape(ECHUNKS, C)
  dstp = jnp.concatenate(
      [dst, jnp.full((EPAD - E,), NPAD, jnp.int32)]).reshape(ECHUNKS, C)
  xp = jnp.zeros((NPAD, D), jnp.float32).at[:N].set(x)
  batp = jnp.concatenate(
      [batch.astype(jnp.int32), jnp.full((NPAD - N,), TRASH, jnp.int32)]
  ).reshape(PCHUNKS, C)
  batp64 = batp + NG

  psrc, pdst, cnts = part_k(srcp, dstp)
  degp = deg_k(pdst, cnts)
  hp1, dinv = _tc_pre(xp, W1, degp)
  parts1 = agg_k(hp1, psrc, pdst, cnts)
  hp2 = _tc_mid(parts1, hp1, dinv, b1.reshape(1, D), W2)
  parts2 = agg_k(hp2, psrc, pdst, cnts)
  hp3 = _tc_mid(parts2, hp2, dinv, b2.reshape(1, D), W3)
  parts3 = agg_k(hp3, psrc, pdst, cnts)
  h3 = _tc_post(parts3, hp3, dinv, b3.reshape(1, D))
  pool = pool_k(h3, batp, batp64)
  return _tc_head(pool, fW1, fb1.reshape(1, D), fW2, fb2.reshape(1, NCLS))


# speedup vs baseline: 24.1213x; 4.5192x over previous
import functools

import jax
import jax.numpy as jnp
from jax import lax
from jax.experimental import pallas as pl
from jax.experimental.pallas import tpu as pltpu
from jax.experimental.pallas import tpu_sc as plsc

N = 10000
E = 320000
D = 128
NCLS = 16
NG = 64

NPAD = 10240
DUMMY = N
C = 128
ECHUNKS = 2560
EPAD = ECHUNKS * C
CPW = 80
HALF = NPAD // 2
ACC_R = HALF + 128
ATRASH = HALF
OPT = HALF // 16
RCH = 88
RCAP = RCH * C
PCHUNKS = NPAD // C
POOL_ROWS = 256
TRASH = 128


def _fill_rows(ref, nrows, value):
  vec = jnp.full((16,), value, jnp.float32)

  def body(i, carry):
    for cc in range(8):
      ref[i, pl.ds(cc * 16, 16)] = vec
    return carry

  lax.fori_loop(0, nrows, body, 0)


def _zero_acc_slice(zeros_v, acc, sid):
  rows = ACC_R // 16
  base = sid * rows
  pltpu.sync_copy(zeros_v, acc.at[pl.ds(base, C)])
  pltpu.sync_copy(zeros_v, acc.at[pl.ds(base + C, C)])
  pltpu.sync_copy(zeros_v.at[pl.ds(0, rows - 2 * C)],
                  acc.at[pl.ds(base + 2 * C, rows - 2 * C)])


def _acc_to_out(acc, out_hbm, cid, sid):
  for o, sz in ((0, C), (C, C), (2 * C, OPT - 2 * C)):
    pltpu.sync_copy(acc.at[pl.ds(sid * OPT + o, sz)],
                    out_hbm.at[pl.ds(cid * HALF + sid * OPT + o, sz)])


def _popcount(mask):
  return jnp.max(plsc.all_reduce_population_count(mask))


def _make_part_kernel(mesh):
  @functools.partial(
      pl.kernel,
      out_type=[
          jax.ShapeDtypeStruct((64 * RCH, C), jnp.int32),
          jax.ShapeDtypeStruct((64 * RCH, C), jnp.int32),
          jax.ShapeDtypeStruct((512,), jnp.int32),
      ],
      mesh=mesh,
      scratch_types=[
          pltpu.VMEM((CPW, C), jnp.int32),
          pltpu.VMEM((CPW, C), jnp.int32),
          [pltpu.VMEM((RCH, C), jnp.int32) for _ in range(2)],
          [pltpu.VMEM((RCH, C), jnp.int32) for _ in range(2)],
          pltpu.VMEM((16,), jnp.int32),
          pltpu.SemaphoreType.DMA,
      ],
      compiler_params=pltpu.CompilerParams(needs_layout_passes=False),
  )
  def part_kernel(src_hbm, dst_hbm, psrc_hbm, pdst_hbm, cnt_hbm,
                  src_v, dst_v, outs, outd, cbuf, sem):
    cid = lax.axis_index("c")
    sid = lax.axis_index("s")
    w = sid * 2 + cid
    pltpu.sync_copy(src_hbm.at[pl.ds(w * CPW, CPW)], src_v)
    pltpu.sync_copy(dst_hbm.at[pl.ds(w * CPW, CPW)], dst_v)

    def body(k, carry):
      c0, c1 = carry
      for cc in range(8):
        s = src_v[k, pl.ds(cc * 16, 16)]
        d = dst_v[k, pl.ds(cc * 16, 16)]
        m0 = d < HALF
        m1 = (d >= HALF) & (d < NPAD)
        cum0 = plsc.cumsum(m0.astype(jnp.int32))
        cum1 = plsc.cumsum(m1.astype(jnp.int32))
        p0 = c0 + cum0 - 1
        p1 = c1 + cum1 - 1
        plsc.store_scatter(outs[0], [p0 >> 7, p0 & 127], s, mask=m0)
        plsc.store_scatter(outd[0], [p0 >> 7, p0 & 127], d, mask=m0)
        plsc.store_scatter(outs[1], [p1 >> 7, p1 & 127], s, mask=m1)
        plsc.store_scatter(outd[1], [p1 >> 7, p1 & 127], d - HALF, mask=m1)
        c0 = c0 + jnp.max(cum0)
        c1 = c1 + jnp.max(cum1)
      return (c0, c1)

    c0, c1 = lax.fori_loop(0, CPW, body, (jnp.int32(0), jnp.int32(0)))

    lane = lax.iota(jnp.int32, 16)
    dummy_vec = jnp.full((16,), DUMMY, jnp.int32)
    for h, cnt in ((0, c0), (1, c1)):
      tgt = ((cnt + 511) // 512) * 512

      def pad_body(i, carry):
        off = cnt + i * 16

        @pl.when(off < tgt)
        def _():
          q = off + lane
          plsc.store_scatter(outs[h], [q >> 7, q & 127], DUMMY + (q & 127))
          plsc.store_scatter(outd[h], [q >> 7, q & 127],
                             ATRASH + ((i * 16 + lane) & 127))

        return carry

      lax.fori_loop(0, 32, pad_body, 0)

      cbuf[...] = jnp.broadcast_to(tgt // 512, (16,)).astype(jnp.int32)
      pltpu.sync_copy(cbuf.at[pl.ds(0, 8)],
                      cnt_hbm.at[pl.ds(w * 16 + h * 8, 8)])
      pltpu.sync_copy(outs[h], psrc_hbm.at[pl.ds((h * 32 + w) * RCH, RCH)])
      pltpu.sync_copy(outd[h], pdst_hbm.at[pl.ds((h * 32 + w) * RCH, RCH)])

  return part_kernel


def _make_deg_kernel(mesh):
  @functools.partial(
      pl.kernel,
      out_type=jax.ShapeDtypeStruct((NPAD, D), jnp.float32),
      mesh=mesh,
      scratch_types=[
          pltpu.VMEM((RCH, C), jnp.int32),
          pltpu.VMEM((16,), jnp.int32),
          pltpu.VMEM((C, D), jnp.float32),
          [pltpu.SemaphoreType.DMA for _ in range(2)],
          pltpu.VMEM_SHARED((ACC_R, D), jnp.float32),
      ],
  )
  def deg_kernel(pdst_hbm, cnt_hbm, out_hbm, dst_v, cnt_v, buf, sems, acc):
    cid = lax.axis_index("c")
    sid = lax.axis_index("s")
    _fill_rows(buf, C, 0.0)
    _zero_acc_slice(buf, acc, sid)
    plsc.subcore_barrier()
    _fill_rows(buf, C, 1.0)

    def scatter(k, b):
      return pltpu.make_async_copy(buf, acc.at[dst_v.at[k]], sems[b])

    for rr in range(2):
      w = 2 * sid + rr
      ridx = cid * 32 + w
      pltpu.sync_copy(pdst_hbm.at[pl.ds(ridx * RCH, RCH)], dst_v)
      pltpu.sync_copy(cnt_hbm.at[pl.ds(w * 16 + cid * 8, 8)],
                      cnt_v.at[pl.ds(0, 8)])
      t4 = cnt_v[...][0]

      @pl.when(t4 > 0)
      def _():
        def body(t, carry):
          for i in range(2):
            k = 2 * t + i

            @pl.when(t > 0)
            def _():
              scatter(k - 2, i).wait()

            scatter(k, i).start(add=True)
          return carry

        lax.fori_loop(0, 2 * t4, body, 0)
        scatter(4 * t4 - 2, 0).wait()
        scatter(4 * t4 - 1, 1).wait()

    plsc.subcore_barrier()
    _acc_to_out(acc, out_hbm, cid, sid)

  return deg_kernel


def _make_agg_kernel(mesh):
  @functools.partial(
      pl.kernel,
      out_type=jax.ShapeDtypeStruct((NPAD, D), jnp.float32),
      mesh=mesh,
      scratch_types=[
          pltpu.VMEM((RCH, C), jnp.int32),
          pltpu.VMEM((RCH, C), jnp.int32),
          pltpu.VMEM((16,), jnp.int32),
          [pltpu.VMEM((C, D), jnp.float32) for _ in range(4)],
          [pltpu.SemaphoreType.DMA for _ in range(4)],
          [pltpu.SemaphoreType.DMA for _ in range(4)],
          pltpu.VMEM_SHARED((ACC_R, D), jnp.float32),
      ],
  )
  def agg_kernel(hp_hbm, psrc_hbm, pdst_hbm, cnt_hbm, out_hbm,
                 src_v, dst_v, cnt_v, rows, gsem, ssem, acc):
    cid = lax.axis_index("c")
    sid = lax.axis_index("s")
    _fill_rows(rows[0], C, 0.0)
    _zero_acc_slice(rows[0], acc, sid)
    plsc.subcore_barrier()

    def gather(k, b):
      return pltpu.make_async_copy(hp_hbm.at[src_v.at[k]], rows[b], gsem[b])

    def scatter(k, b):
      return pltpu.make_async_copy(rows[b], acc.at[dst_v.at[k]], ssem[b])

    for rr in range(2):
      w = 2 * sid + rr
      ridx = cid * 32 + w
      pltpu.sync_copy(psrc_hbm.at[pl.ds(ridx * RCH, RCH)], src_v)
      pltpu.sync_copy(pdst_hbm.at[pl.ds(ridx * RCH, RCH)], dst_v)
      pltpu.sync_copy(cnt_hbm.at[pl.ds(w * 16 + cid * 8, 8)],
                      cnt_v.at[pl.ds(0, 8)])
      t4 = cnt_v[...][0]

      @pl.when(t4 > 0)
      def _():
        gather(0, 0).start()
        gather(1, 1).start()

        def body(t, carry):
          for i in range(4):
            k = 4 * t + i
            b = i
            bn = (i + 2) % 4
            if i < 2:
              @pl.when(t > 0)
              def _():
                scatter(k - 2, bn).wait()

              gather(k + 2, bn).start()
            else:
              scatter(k - 2, bn).wait()

              @pl.when(t < t4 - 1)
              def _():
                gather(k + 2, bn).start()

            gather(k, b).wait()
            scatter(k, b).start(add=True)
          return carry

        lax.fori_loop(0, t4, body, 0)
        scatter(4 * t4 - 2, 2).wait()
        scatter(4 * t4 - 1, 3).wait()

    plsc.subcore_barrier()
    _acc_to_out(acc, out_hbm, cid, sid)

  return agg_kernel


def _make_pool_kernel(mesh):
  @functools.partial(
      pl.kernel,
      out_type=jax.ShapeDtypeStruct((2, POOL_ROWS // 2, D), jnp.float32),
      mesh=mesh,
      scratch_types=[
          pltpu.VMEM((1, C), jnp.int32),
          pltpu.VMEM((1, C), jnp.int32),
          pltpu.VMEM((C, D), jnp.float32),
          pltpu.VMEM((C, D), jnp.float32),
          pltpu.VMEM((16, D), jnp.float32),
          pltpu.VMEM_SHARED((POOL_ROWS, D), jnp.float32),
      ],
  )
  def pool_kernel(h3_hbm, bat_hbm, bat64_hbm, out_hbm,
                  bidx, b64idx, rows_v, ones_v, zeros_v, acc):
    cid = lax.axis_index("c")
    sid = lax.axis_index("s")
    w = sid * 2 + cid
    _fill_rows(ones_v, C, 1.0)
    _fill_rows(zeros_v, 16, 0.0)
    pltpu.sync_copy(zeros_v, acc.at[pl.ds(sid * 16, 16)])
    plsc.subcore_barrier()
    for j in range(3):
      c = w + 32 * j

      @pl.when(c < PCHUNKS)
      def _():
        pltpu.sync_copy(bat_hbm.at[pl.ds(c, 1)], bidx)
        pltpu.sync_copy(bat64_hbm.at[pl.ds(c, 1)], b64idx)
        pltpu.sync_copy(h3_hbm.at[pl.ds(c * C, C)], rows_v)
        pltpu.sync_copy(rows_v, acc.at[bidx.at[0]], add=True)
        pltpu.sync_copy(ones_v, acc.at[b64idx.at[0]], add=True)

    plsc.subcore_barrier()
    pltpu.sync_copy(acc.at[pl.ds(sid * 8, 8)],
                    out_hbm.at[cid, pl.ds(sid * 8, 8)])

  return pool_kernel


def _tc_z1_body(x_ref, w_ref, z_ref):
  z_ref[...] = jnp.dot(
      x_ref[...], w_ref[...], preferred_element_type=jnp.float32)


def _tc_pre_body(z_ref, degp_ref, hp_ref, dinv_ref):
  deg = degp_ref[...] + 1.0
  dinv = lax.rsqrt(deg)
  hp_ref[...] = dinv * z_ref[...]
  dinv_ref[...] = dinv[:, 0:1]


def _tc_mid_body(p_ref, hp_ref, dinv_ref, b_ref, w_ref, out_ref):
  agg = p_ref[...] + hp_ref[...]
  h = jnp.maximum(dinv_ref[...] * agg + b_ref[...], 0.0)
  out_ref[...] = dinv_ref[...] * jnp.dot(
      h, w_ref[...], preferred_element_type=jnp.float32)


def _tc_post_body(p_ref, hp_ref, dinv_ref, b_ref, out_ref):
  agg = p_ref[...] + hp_ref[...]
  out_ref[...] = jnp.maximum(dinv_ref[...] * agg + b_ref[...], 0.0)


def _tc_head_body(pp_ref, fw1_ref, fb1_ref, fw2_ref, fb2_ref, out_ref):
  p = pp_ref[0] + pp_ref[1]
  s = p[0:NG, :]
  cnt = p[NG:2 * NG, 0:1]
  g = s / jnp.maximum(cnt, 1.0)
  r = jnp.maximum(
      jnp.dot(g, fw1_ref[...], preferred_element_type=jnp.float32)
      + fb1_ref[...], 0.0)
  out_ref[...] = jnp.dot(
      r, fw2_ref[...], preferred_element_type=jnp.float32) + fb2_ref[...]


_R = 1024
_G = NPAD // _R


def _row_spec(bl=_R):
  return pl.BlockSpec((bl, D), lambda i: (i, 0))


def _tc_z1(xp, W1):
  return pl.pallas_call(
      _tc_z1_body,
      grid=(_G,),
      in_specs=[
          _row_spec(),
          pl.BlockSpec((D, D), lambda i: (0, 0)),
      ],
      out_specs=_row_spec(),
      out_shape=jax.ShapeDtypeStruct((NPAD, D), jnp.float32),
  )(xp, W1)


def _tc_pre(z1, degp):
  return pl.pallas_call(
      _tc_pre_body,
      grid=(_G,),
      in_specs=[
          _row_spec(),
          _row_spec(),
      ],
      out_specs=[
          _row_spec(),
          pl.BlockSpec((_R, 1), lambda i: (i, 0)),
      ],
      out_shape=[
          jax.ShapeDtypeStruct((NPAD, D), jnp.float32),
          jax.ShapeDtypeStruct((NPAD, 1), jnp.float32),
      ],
  )(z1, degp)


def _tc_mid(parts, hp, dinv, b, W):
  return pl.pallas_call(
      _tc_mid_body,
      grid=(_G,),
      in_specs=[
          _row_spec(),
          _row_spec(),
          pl.BlockSpec((_R, 1), lambda i: (i, 0)),
          pl.BlockSpec((1, D), lambda i: (0, 0)),
          pl.BlockSpec((D, D), lambda i: (0, 0)),
      ],
      out_specs=_row_spec(),
      out_shape=jax.ShapeDtypeStruct((NPAD, D), jnp.float32),
  )(parts, hp, dinv, b, W)


def _tc_post(parts, hp, dinv, b):
  return pl.pallas_call(
      _tc_post_body,
      grid=(_G,),
      in_specs=[
          _row_spec(),
          _row_spec(),
          pl.BlockSpec((_R, 1), lambda i: (i, 0)),
          pl.BlockSpec((1, D), lambda i: (0, 0)),
      ],
      out_specs=_row_spec(),
      out_shape=jax.ShapeDtypeStruct((NPAD, D), jnp.float32),
  )(parts, hp, dinv, b)


def _tc_head(pool, fW1, fb1, fW2, fb2):
  return pl.pallas_call(
      _tc_head_body,
      out_shape=jax.ShapeDtypeStruct((NG, NCLS), jnp.float32),
  )(pool, fW1, fb1, fW2, fb2)


def kernel(x, edge_index, batch, W1, b1, W2, b2, W3, b3, fW1, fb1, fW2, fb2):
  mesh = plsc.VectorSubcoreMesh(core_axis_name="c", subcore_axis_name="s")
  part_k = _make_part_kernel(mesh)
  deg_k = _make_deg_kernel(mesh)
  agg_k = _make_agg_kernel(mesh)
  pool_k = _make_pool_kernel(mesh)

  src = edge_index[0].astype(jnp.int32)
  dst = edge_index[1].astype(jnp.int32)
  srcp = jnp.concatenate(
      [src, DUMMY + (jnp.arange(EPAD - E, dtype=jnp.int32) & 127)]
  ).reshape(ECHUNKS, C)
  dstp = jnp.concatenate(
      [dst, jnp.full((EPAD - E,), NPAD, jnp.int32)]).reshape(ECHUNKS, C)
  xp = jnp.zeros((NPAD, D), jnp.float32).at[:N].set(x)
  batp = jnp.concatenate(
      [batch.astype(jnp.int32), jnp.full((NPAD - N,), TRASH, jnp.int32)]
  ).reshape(PCHUNKS, C)
  batp64 = batp + NG

  z1 = _tc_z1(xp, W1)
  psrc, pdst, cnts = part_k(srcp, dstp)
  degp = deg_k(pdst, cnts)
  hp1, dinv = _tc_pre(z1, degp)
  parts1 = agg_k(hp1, psrc, pdst, cnts)
  hp2 = _tc_mid(parts1, hp1, dinv, b1.reshape(1, D), W2)
  parts2 = agg_k(hp2, psrc, pdst, cnts)
  hp3 = _tc_mid(parts2, hp2, dinv, b2.reshape(1, D), W3)
  parts3 = agg_k(hp3, psrc, pdst, cnts)
  h3 = _tc_post(parts3, hp3, dinv, b3.reshape(1, D))
  pool = pool_k(h3, batp, batp64)
  return _tc_head(pool, fW1, fb1.reshape(1, D), fW2, fb2.reshape(1, NCLS))
